# baseline (speedup 1.0000x reference)
import math

import jax
import jax.numpy as jnp
from jax import lax
from jax.experimental import pallas as pl
from jax.experimental.pallas import tpu as pltpu
from jax.experimental.pallas import tpu_sc as plsc

NC = 2
NS = 16
NW = NC * NS
L = 16


def _sc_partials(x, src_r, dst_r, acc_rows, ch, k, csb, depth):
    n, d = x.shape
    zt = acc_rows // NS
    mesh = plsc.VectorSubcoreMesh(core_axis_name="c", subcore_axis_name="s")
    zc = (k // 8) * 8
    zoffs = [(off, min(zc, zt - off)) for off in range(0, zt, zc)]

    lt = n - (NS - 1) * zt

    def body(x_hbm, src_hbm, dst_hbm, zeros_hbm, out_hbm, src_v, dst_v, *rest):
        bufs = rest[:depth]
        acc = rest[depth]
        sems = rest[depth + 1:]
        cid = lax.axis_index("c")
        sid = lax.axis_index("s")
        wid = sid * NC + cid

        pltpu.sync_copy(src_hbm.at[wid, pl.ds(0, csb)], src_v)
        pltpu.sync_copy(dst_hbm.at[wid, pl.ds(0, csb)], dst_v)
        for b in range(depth):
            pltpu.async_copy(x_hbm.at[src_v.at[b]], bufs[b], sems[b])

        zbase = pl.multiple_of(sid * zt, 8)
        pltpu.sync_copy(zeros_hbm, acc.at[pl.ds(zbase, zt)])
        plsc.subcore_barrier()

        for sb in range(ch // csb):
            if sb:
                pltpu.sync_copy(src_hbm.at[wid, pl.ds(sb * csb, csb)], src_v)
                pltpu.sync_copy(dst_hbm.at[wid, pl.ds(sb * csb, csb)], dst_v)
                for b in range(depth):
                    pltpu.async_copy(x_hbm.at[src_v.at[b]], bufs[b], sems[b])

            def step(i, carry):
                j0 = depth * i
                for b in range(depth):
                    j = j0 + b
                    pltpu.make_async_copy(x_hbm.at[src_v.at[j]],
                                          bufs[b], sems[b]).wait()
                    pltpu.sync_copy(bufs[b], acc.at[dst_v.at[j]], add=True)

                    @pl.when(j + depth < csb)
                    def _():
                        pltpu.async_copy(x_hbm.at[src_v.at[j + depth]],
                                         bufs[b], sems[b])
                return carry

            lax.fori_loop(0, csb // depth, step, 0)
        plsc.subcore_barrier()

        @pl.when(sid < NS - 1)
        def _():
            pltpu.sync_copy(acc.at[pl.ds(zbase, zt)],
                            out_hbm.at[cid, pl.ds(sid * zt, zt)])

        @pl.when(sid == NS - 1)
        def _():
            pltpu.sync_copy(acc.at[pl.ds(zbase, lt)],
                            out_hbm.at[cid, pl.ds((NS - 1) * zt, lt)])

    call = pl.kernel(
        body,
        out_type=jax.ShapeDtypeStruct((NC, n, d), jnp.float32),
        mesh=mesh,
        scratch_types=(
            [pltpu.VMEM((csb, k), jnp.int32),
             pltpu.VMEM((csb, k), jnp.int32)]
            + [pltpu.VMEM((k, d), jnp.float32) for _ in range(depth)]
            + [pltpu.VMEM_SHARED((acc_rows, d), jnp.float32)]
            + [pltpu.SemaphoreType.DMA for _ in range(depth)]
        ),
    )
    return call(x, src_r, dst_r, jnp.zeros((zt, d), jnp.float32))


def _combine(p0, p1, nn, n):
    d = p0.shape[1]
    r = next((b for b in (2000, 1000, 400, 200, 80, 40, 16, 8) if n % b == 0), n)

    def body(nn_ref, a_ref, b_ref, o_ref):
        i = pl.program_id(0)
        rows = lax.broadcasted_iota(jnp.int32, (r, d), 0) + i * r
        s = a_ref[...] + b_ref[...]
        o_ref[...] = jnp.where(rows < nn_ref[0], s, 0.0)

    return pl.pallas_call(
        body,
        grid=(n // r,),
        in_specs=[
            pl.BlockSpec(memory_space=pltpu.SMEM),
            pl.BlockSpec((r, d), lambda i: (i, 0)),
            pl.BlockSpec((r, d), lambda i: (i, 0)),
        ],
        out_specs=pl.BlockSpec((r, d), lambda i: (i, 0)),
        out_shape=jax.ShapeDtypeStruct((n, d), jnp.float32),
    )(nn, p0, p1)


def kernel(x, edge_index, num_nodes):
    n, d = x.shape
    e = edge_index.shape[1]
    ei = edge_index.astype(jnp.int32)

    acc_rows = ((n + NS * 8 - 1) // (NS * 8)) * (NS * 8)

    budget = 2097151 - acc_rows * d - 30000
    best = None
    for k in range(128, 7, -1):
        if e % (NW * k):
            continue
        ch = e // (NW * k)
        bufw = ((k + 7) // 8) * 8 * (((d + 127) // 128) * 128)
        for depth in range(6, 1, -1):
            for csb in range(min(48, ch), 0, -1):
                if ch % csb or csb % depth:
                    continue
                idxw = ((csb + 7) // 8) * 8 * (((k + 127) // 128) * 128)
                if NS * (2 * idxw + depth * bufw) <= budget:
                    if best is None or (k, depth) > (best[1], best[0]):
                        best = (depth, k, csb)
                    break
    depth, k, csb = best
    ch = e // (NW * k)
    er = ei.reshape(2, NW, ch, k)

    p = _sc_partials(x, er[0], er[1], acc_rows, ch, k, csb, depth)
    nn = jnp.reshape(num_nodes, (1,)).astype(jnp.int32)
    return _combine(p[0], p[1], nn, n)


# speedup vs baseline: 1.0175x; 1.0175x over previous
import math

import jax
import jax.numpy as jnp
from jax import lax
from jax.experimental import pallas as pl
from jax.experimental.pallas import tpu as pltpu
from jax.experimental.pallas import tpu_sc as plsc

NC = 2
NS = 16
NW = NC * NS
L = 16


def _sc_partials(x, src_r, dst_r, acc_rows, ch, k, csb, depth):
    n, d = x.shape
    zt = acc_rows // NS
    mesh = plsc.VectorSubcoreMesh(core_axis_name="c", subcore_axis_name="s")
    zc = (k // 8) * 8
    zoffs = [(off, min(zc, zt - off)) for off in range(0, zt, zc)]

    lt = n - (NS - 1) * zt

    def body(x_hbm, src_hbm, dst_hbm, out_hbm, src_v, dst_v, *rest):
        bufs = rest[:depth]
        acc = rest[depth]
        sems = rest[depth + 1:]
        cid = lax.axis_index("c")
        sid = lax.axis_index("s")
        wid = sid * NC + cid

        zero = jnp.zeros((L,), jnp.float32)

        def zrow(j, carry):
            for t in range(0, d, L):
                bufs[0][j, pl.ds(t, L)] = zero
            return carry

        lax.fori_loop(0, k, zrow, 0)
        zbase = pl.multiple_of(sid * zt, 8)
        for off, sz in zoffs:
            pltpu.sync_copy(bufs[0].at[pl.ds(0, sz)],
                            acc.at[pl.ds(zbase + off, sz)])
        plsc.subcore_barrier()

        for sb in range(ch // csb):
            pltpu.sync_copy(src_hbm.at[wid, pl.ds(sb * csb, csb)], src_v)
            pltpu.sync_copy(dst_hbm.at[wid, pl.ds(sb * csb, csb)], dst_v)
            for b in range(depth):
                pltpu.async_copy(x_hbm.at[src_v.at[b]], bufs[b], sems[b])

            def step(i, carry):
                j0 = depth * i
                for b in range(depth):
                    j = j0 + b
                    pltpu.make_async_copy(x_hbm.at[src_v.at[j]],
                                          bufs[b], sems[b]).wait()
                    pltpu.sync_copy(bufs[b], acc.at[dst_v.at[j]], add=True)

                    @pl.when(j + depth < csb)
                    def _():
                        pltpu.async_copy(x_hbm.at[src_v.at[j + depth]],
                                         bufs[b], sems[b])
                return carry

            lax.fori_loop(0, csb // depth, step, 0)
        plsc.subcore_barrier()

        @pl.when(sid < NS - 1)
        def _():
            pltpu.sync_copy(acc.at[pl.ds(zbase, zt)],
                            out_hbm.at[cid, pl.ds(sid * zt, zt)])

        @pl.when(sid == NS - 1)
        def _():
            pltpu.sync_copy(acc.at[pl.ds(zbase, lt)],
                            out_hbm.at[cid, pl.ds((NS - 1) * zt, lt)])

    call = pl.kernel(
        body,
        out_type=jax.ShapeDtypeStruct((NC, n, d), jnp.float32),
        mesh=mesh,
        scratch_types=(
            [pltpu.VMEM((csb, k), jnp.int32),
             pltpu.VMEM((csb, k), jnp.int32)]
            + [pltpu.VMEM((k, d), jnp.float32) for _ in range(depth)]
            + [pltpu.VMEM_SHARED((acc_rows, d), jnp.float32)]
            + [pltpu.SemaphoreType.DMA for _ in range(depth)]
        ),
    )
    return call(x, src_r, dst_r)


def _combine(p0, p1, nn, n):
    d = p0.shape[1]
    r = next((b for b in (2000, 1000, 400, 200, 80, 40, 16, 8) if n % b == 0), n)

    def body(nn_ref, a_ref, b_ref, o_ref):
        i = pl.program_id(0)
        rows = lax.broadcasted_iota(jnp.int32, (r, d), 0) + i * r
        s = a_ref[...] + b_ref[...]
        o_ref[...] = jnp.where(rows < nn_ref[0], s, 0.0)

    return pl.pallas_call(
        body,
        grid=(n // r,),
        in_specs=[
            pl.BlockSpec(memory_space=pltpu.SMEM),
            pl.BlockSpec((r, d), lambda i: (i, 0)),
            pl.BlockSpec((r, d), lambda i: (i, 0)),
        ],
        out_specs=pl.BlockSpec((r, d), lambda i: (i, 0)),
        out_shape=jax.ShapeDtypeStruct((n, d), jnp.float32),
    )(nn, p0, p1)


def kernel(x, edge_index, num_nodes):
    n, d = x.shape
    e = edge_index.shape[1]
    ei = edge_index.astype(jnp.int32)

    acc_rows = ((n + NS * 8 - 1) // (NS * 8)) * (NS * 8)

    budget = 2097151 - acc_rows * d - 30000
    best = None
    for k in range(128, 7, -1):
        if e % (NW * k):
            continue
        ch = e // (NW * k)
        bufw = ((k + 7) // 8) * 8 * (((d + 127) // 128) * 128)
        for depth in range(6, 1, -1):
            for csb in range(min(48, ch), 0, -1):
                if ch % csb or csb % depth:
                    continue
                idxw = ((csb + 7) // 8) * 8 * (((k + 127) // 128) * 128)
                if NS * (2 * idxw + depth * bufw) <= budget:
                    if best is None or (k, depth) > (best[1], best[0]):
                        best = (depth, k, csb)
                    break
    depth, k, csb = best
    ch = e // (NW * k)
    er = ei.reshape(2, NW, ch, k)

    p = _sc_partials(x, er[0], er[1], acc_rows, ch, k, csb, depth)
    nn = jnp.reshape(num_nodes, (1,)).astype(jnp.int32)
    return _combine(p[0], p[1], nn, n)
